# baseline (speedup 1.0000x reference)
import functools

import jax
import jax.numpy as jnp
import numpy as np
from jax.experimental import pallas as pl
from jax.experimental.pallas import tpu as pltpu

K_N = 2048
BATCH = 2
D_MODEL = 768
N_HEADS = 12
HEAD_DIM = D_MODEL // N_HEADS
FACTOR = 5
U_TOP = int(FACTOR * np.ceil(np.log(K_N)))
PAD_U = 64
ROW_BLK = 256
N_ROW_BLKS = K_N // ROW_BLK
NEG_BIG = -1e30

def _np_rotl(x, d):
    return ((x << np.uint32(d)) | (x >> np.uint32(32 - d))).astype(np.uint32)


def _np_threefry2x32(k1, k2, x0, x1):
    ks = [np.uint32(k1), np.uint32(k2),
          np.uint32(k1 ^ k2 ^ np.uint32(0x1BD11BDA))]
    x0 = (x0 + ks[0]).astype(np.uint32)
    x1 = (x1 + ks[1]).astype(np.uint32)
    rotations = [[13, 15, 26, 6], [17, 29, 16, 24]]
    for i in range(5):
        for r in rotations[i % 2]:
            x0 = (x0 + x1).astype(np.uint32)
            x1 = _np_rotl(x1, r) ^ x0
        x0 = (x0 + ks[(i + 1) % 3]).astype(np.uint32)
        x1 = (x1 + ks[(i + 2) % 3] + np.uint32(i + 1)).astype(np.uint32)
    return x0, x1


def _np_randint_key42(shape, span):
    b1, b2 = _np_threefry2x32(np.uint32(0), np.uint32(42),
                              np.array([0, 0], np.uint32),
                              np.array([0, 1], np.uint32))
    n = int(np.prod(shape))
    idx = np.arange(n, dtype=np.uint64)
    c1 = (idx >> np.uint64(32)).astype(np.uint32)
    c2 = (idx & np.uint64(0xFFFFFFFF)).astype(np.uint32)
    h1, h2 = _np_threefry2x32(b1[0], b2[0], c1, c2)
    l1, l2 = _np_threefry2x32(b1[1], b2[1], c1, c2)
    hi, lo = h1 ^ h2, l1 ^ l2
    mult = (np.uint32(2 ** 16) % np.uint32(span))
    mult = (mult * mult) % np.uint32(span)
    off = ((hi % np.uint32(span)) * mult + lo % np.uint32(span)) \
        % np.uint32(span)
    return off.reshape(shape).astype(np.int32)


_INDEX_SAMPLE = _np_randint_key42((K_N, U_TOP), K_N)
_CNT_T = np.zeros((K_N, K_N), np.float32)
np.add.at(_CNT_T, (_INDEX_SAMPLE, np.arange(K_N)[:, None]), 1.0)
_CNT_T_BF16 = _CNT_T.astype(jnp.bfloat16)

def _dot(a, b, dimension_numbers):
    return jax.lax.dot_general(
        a.astype(jnp.bfloat16), b.astype(jnp.bfloat16), dimension_numbers,
        preferred_element_type=jnp.float32)


def _body(q_ref, k_ref, v_ref, wq_ref, wk_ref, wv_ref, bq_ref, bk_ref,
          bv_ref, wo_ref, bo_ref, cnt_ref, out_ref,
          qh_s, kh_s, vh_s, m_s, qidx_s):
    scaling = HEAD_DIM ** -0.5
    xq = q_ref[0]
    xk = k_ref[0]
    xv = v_ref[0]

    row_io = jax.lax.broadcasted_iota(jnp.int32, (N_ROW_BLKS, ROW_BLK), 0)
    col_io = jax.lax.broadcasted_iota(jnp.int32, (N_ROW_BLKS, ROW_BLK), 1)
    flat_io = row_io * ROW_BLK + col_io
    sel_io = jax.lax.broadcasted_iota(jnp.int32, (PAD_U, K_N), 1)
    urow_io = jax.lax.broadcasted_iota(jnp.int32, (PAD_U, HEAD_DIM), 0)

    qidx_s[...] = jnp.full((PAD_U, 1), -1.0, jnp.float32)
    base = None

    for h in range(N_HEADS):
        sl = slice(h * HEAD_DIM, (h + 1) * HEAD_DIM)
        qf = (_dot(xq, wq_ref[sl, :],
                   dimension_numbers=(((1,), (1,)), ((), ()))) +
              bq_ref[:, sl]) * scaling
        qh_s[...] = qf
        kf = _dot(xk, wk_ref[sl, :],
                  dimension_numbers=(((1,), (1,)), ((), ()))) + bk_ref[:, sl]
        kh_s[...] = kf.astype(jnp.bfloat16)
        vh_s[...] = _dot(xv, wv_ref[sl, :],
                         dimension_numbers=(((1,), (1,)), ((), ()))) + \
            bv_ref[:, sl]

        k_bf = kh_s[...]
        for rb in range(N_ROW_BLKS):
            q_blk = qh_s[rb * ROW_BLK:(rb + 1) * ROW_BLK, :]
            s_t = _dot(k_bf, q_blk,
                       dimension_numbers=(((1,), (1,)), ((), ())))
            cf = cnt_ref[:, rb * ROW_BLK:(rb + 1) * ROW_BLK].astype(
                jnp.float32)
            mx = jnp.max(jnp.where(cf > 0.0, s_t, NEG_BIG), axis=0)
            sm = jnp.sum(s_t * cf, axis=0)
            m_s[rb:rb + 1, :] = (mx - sm * (1.0 / K_N)).reshape(1, ROW_BLK)

        def pick(u, carry):
            mcur = m_s[...]
            mval = jnp.max(mcur)
            i = jnp.min(jnp.where(mcur == mval, flat_io, K_N))
            qidx_s[pl.ds(u, 1), :] = i.astype(jnp.float32).reshape(1, 1)
            m_s[...] = jnp.where(flat_io == i, NEG_BIG, mcur)
            return carry

        jax.lax.fori_loop(0, U_TOP, pick, 0)

        qidx = qidx_s[...].astype(jnp.int32)
        sel = jnp.where(sel_io == qidx, 1.0, 0.0)

        qsel = _dot(sel, qh_s[...],
                    dimension_numbers=(((1,), (0,)), ((), ())))
        scores = _dot(qsel, k_bf,
                      dimension_numbers=(((1,), (1,)), ((), ())))
        scores_max = jnp.max(scores, axis=1, keepdims=True)
        e = jnp.exp(scores - scores_max)
        attn = e / jnp.sum(e, axis=1, keepdims=True)
        v = vh_s[...]
        upd = _dot(attn, v, dimension_numbers=(((1,), (0,)), ((), ())))
        mean_v = jnp.mean(v, axis=0, keepdims=True)
        updm = jnp.where(urow_io == U_TOP, mean_v, upd)

        p = _dot(updm, wo_ref[sl, :],
                 dimension_numbers=(((1,), (0,)), ((), ())))

        pmean = p[U_TOP:U_TOP + 1, :]
        base = pmean if base is None else base + pmean

        corr = _dot(sel, p - pmean,
                    dimension_numbers=(((0,), (0,)), ((), ())))
        if h == 0:
            out_ref[...] = corr[None]
        else:
            out_ref[...] += corr[None]

    out_ref[...] += (base + bo_ref[...])[None]


def kernel(query, key, value, W_Q, b_Q, W_K, b_K, W_V, b_V, W_O, b_O):
    k_N, Bq, d_model = query.shape

    cnt_t = jnp.asarray(_CNT_T_BF16)
    qb = query.transpose(1, 0, 2).astype(jnp.bfloat16)
    kb = key.transpose(1, 0, 2).astype(jnp.bfloat16)
    vb = value.transpose(1, 0, 2).astype(jnp.bfloat16)
    wqb = W_Q.astype(jnp.bfloat16)
    wkb = W_K.astype(jnp.bfloat16)
    wvb = W_V.astype(jnp.bfloat16)
    wob = W_O.T.astype(jnp.bfloat16)
    bq2 = b_Q.reshape(1, d_model)
    bk2 = b_K.reshape(1, d_model)
    bv2 = b_V.reshape(1, d_model)
    bo2 = b_O.reshape(1, d_model)

    out = pl.pallas_call(
        _body,
        grid=(Bq,),
        in_specs=[
            pl.BlockSpec((1, K_N, D_MODEL), lambda b: (b, 0, 0)),
            pl.BlockSpec((1, K_N, D_MODEL), lambda b: (b, 0, 0)),
            pl.BlockSpec((1, K_N, D_MODEL), lambda b: (b, 0, 0)),
            pl.BlockSpec((D_MODEL, D_MODEL), lambda b: (0, 0)),
            pl.BlockSpec((D_MODEL, D_MODEL), lambda b: (0, 0)),
            pl.BlockSpec((D_MODEL, D_MODEL), lambda b: (0, 0)),
            pl.BlockSpec((1, D_MODEL), lambda b: (0, 0)),
            pl.BlockSpec((1, D_MODEL), lambda b: (0, 0)),
            pl.BlockSpec((1, D_MODEL), lambda b: (0, 0)),
            pl.BlockSpec((D_MODEL, D_MODEL), lambda b: (0, 0)),
            pl.BlockSpec((1, D_MODEL), lambda b: (0, 0)),
            pl.BlockSpec((K_N, K_N), lambda b: (0, 0)),
        ],
        out_specs=pl.BlockSpec((1, K_N, D_MODEL), lambda b: (b, 0, 0)),
        out_shape=jax.ShapeDtypeStruct((Bq, k_N, d_model), jnp.float32),
        scratch_shapes=[
            pltpu.VMEM((K_N, HEAD_DIM), jnp.float32),
            pltpu.VMEM((K_N, HEAD_DIM), jnp.bfloat16),
            pltpu.VMEM((K_N, HEAD_DIM), jnp.float32),
            pltpu.VMEM((N_ROW_BLKS, ROW_BLK), jnp.float32),
            pltpu.VMEM((PAD_U, 1), jnp.float32),
        ],
    )(qb, kb, vb, wqb, wkb, wvb, bq2, bk2, bv2, wob, bo2, cnt_t)
    return out


# speedup vs baseline: 1.6528x; 1.6528x over previous
import functools

import jax
import jax.numpy as jnp
import numpy as np
from jax.experimental import pallas as pl
from jax.experimental.pallas import tpu as pltpu

K_N = 2048
BATCH = 2
D_MODEL = 768
N_HEADS = 12
HEAD_DIM = D_MODEL // N_HEADS
FACTOR = 5
U_TOP = int(FACTOR * np.ceil(np.log(K_N)))
PAD_U = 64
ROW_BLK = 256
N_ROW_BLKS = K_N // ROW_BLK
NEG_BIG = -1e30

def _np_rotl(x, d):
    return ((x << np.uint32(d)) | (x >> np.uint32(32 - d))).astype(np.uint32)


def _np_threefry2x32(k1, k2, x0, x1):
    ks = [np.uint32(k1), np.uint32(k2),
          np.uint32(k1 ^ k2 ^ np.uint32(0x1BD11BDA))]
    x0 = (x0 + ks[0]).astype(np.uint32)
    x1 = (x1 + ks[1]).astype(np.uint32)
    rotations = [[13, 15, 26, 6], [17, 29, 16, 24]]
    for i in range(5):
        for r in rotations[i % 2]:
            x0 = (x0 + x1).astype(np.uint32)
            x1 = _np_rotl(x1, r) ^ x0
        x0 = (x0 + ks[(i + 1) % 3]).astype(np.uint32)
        x1 = (x1 + ks[(i + 2) % 3] + np.uint32(i + 1)).astype(np.uint32)
    return x0, x1


def _np_randint_key42(shape, span):
    b1, b2 = _np_threefry2x32(np.uint32(0), np.uint32(42),
                              np.array([0, 0], np.uint32),
                              np.array([0, 1], np.uint32))
    n = int(np.prod(shape))
    idx = np.arange(n, dtype=np.uint64)
    c1 = (idx >> np.uint64(32)).astype(np.uint32)
    c2 = (idx & np.uint64(0xFFFFFFFF)).astype(np.uint32)
    h1, h2 = _np_threefry2x32(b1[0], b2[0], c1, c2)
    l1, l2 = _np_threefry2x32(b1[1], b2[1], c1, c2)
    hi, lo = h1 ^ h2, l1 ^ l2
    mult = (np.uint32(2 ** 16) % np.uint32(span))
    mult = (mult * mult) % np.uint32(span)
    off = ((hi % np.uint32(span)) * mult + lo % np.uint32(span)) \
        % np.uint32(span)
    return off.reshape(shape).astype(np.int32)


_INDEX_SAMPLE = _np_randint_key42((K_N, U_TOP), K_N)
_CNT_T = np.zeros((K_N, K_N), np.float32)
np.add.at(_CNT_T, (_INDEX_SAMPLE, np.arange(K_N)[:, None]), 1.0)
_CNT_T_BF16 = _CNT_T.astype(jnp.bfloat16)

def _dot(a, b, dimension_numbers):
    return jax.lax.dot_general(
        a.astype(jnp.bfloat16), b.astype(jnp.bfloat16), dimension_numbers,
        preferred_element_type=jnp.float32)


def _lane_cumsum(x):
    n = x.shape[-1]
    d = 1
    while d < n:
        x = x + jnp.pad(x, ((0, 0), (d, 0)))[:, :n]
        d *= 2
    return x


def _body(q_ref, k_ref, v_ref, wq_ref, wk_ref, wv_ref, bq_ref, bk_ref,
          bv_ref, wo_ref, bo_ref, cnt_ref, out_ref,
          qh_s, kh_s, vh_s, m_s, base_s):
    h = pl.program_id(1)
    scaling = HEAD_DIM ** -0.5
    xq = q_ref[0]
    xk = k_ref[0]
    xv = v_ref[0]

    urow_io = jax.lax.broadcasted_iota(jnp.int32, (PAD_U, HEAD_DIM), 0)
    ucol_io = jax.lax.broadcasted_iota(
        jnp.int32, (PAD_U, 1), 0).astype(jnp.float32)

    qf = (_dot(xq, wq_ref[0], dimension_numbers=(((1,), (1,)), ((), ()))) +
          bq_ref[0]) * scaling
    qh_s[...] = qf
    kf = _dot(xk, wk_ref[0],
              dimension_numbers=(((1,), (1,)), ((), ()))) + bk_ref[0]
    kh_s[...] = kf.astype(jnp.bfloat16)
    vh_s[...] = _dot(xv, wv_ref[0],
                     dimension_numbers=(((1,), (1,)), ((), ()))) + bv_ref[0]

    k_bf = kh_s[...]
    for rb in range(N_ROW_BLKS):
        q_blk = qh_s[rb * ROW_BLK:(rb + 1) * ROW_BLK, :]
        s_t = _dot(k_bf, q_blk, dimension_numbers=(((1,), (1,)), ((), ())))
        cf = cnt_ref[:, rb * ROW_BLK:(rb + 1) * ROW_BLK].astype(jnp.float32)
        mx = jnp.max(jnp.where(cf > 0.0, s_t, NEG_BIG), axis=0)
        sm = jnp.sum(s_t * cf, axis=0)
        m_s[0:1, rb * ROW_BLK:(rb + 1) * ROW_BLK] = \
            (mx - sm * (1.0 / K_N)).reshape(1, ROW_BLK)

    mrow = m_s[...]
    rank = jnp.zeros((1, K_N), jnp.float32)
    for jb in range(N_ROW_BLKS):
        mjc = mrow[0:1, jb * ROW_BLK:(jb + 1) * ROW_BLK].reshape(ROW_BLK, 1)
        rank = rank + jnp.sum(
            jnp.where(mjc > mrow, 1.0, 0.0), axis=0, keepdims=True)
    vb = jnp.min(jnp.where(rank < float(U_TOP), mrow, jnp.inf),
                 axis=1, keepdims=True)
    gtm = mrow > vb
    eqm = mrow == vb
    need = float(U_TOP) - jnp.sum(jnp.where(gtm, 1.0, 0.0), axis=1,
                                  keepdims=True)
    eqpos = _lane_cumsum(jnp.where(eqm, 1.0, 0.0))
    selm = gtm | (eqm & (eqpos <= need))

    pos = _lane_cumsum(jnp.where(selm, 1.0, 0.0)) - 1.0
    sel = jnp.where((pos == ucol_io) & selm, 1.0, 0.0)

    qsel = _dot(sel, qh_s[...], dimension_numbers=(((1,), (0,)), ((), ())))
    scores = _dot(qsel, k_bf, dimension_numbers=(((1,), (1,)), ((), ())))
    scores_max = jnp.max(scores, axis=1, keepdims=True)
    e = jnp.exp(scores - scores_max)
    attn = e / jnp.sum(e, axis=1, keepdims=True)
    v = vh_s[...]
    upd = _dot(attn, v, dimension_numbers=(((1,), (0,)), ((), ())))
    mean_v = jnp.mean(v, axis=0, keepdims=True)
    updm = jnp.where(urow_io == U_TOP, mean_v, upd)

    p = _dot(updm, wo_ref[0], dimension_numbers=(((1,), (0,)), ((), ())))

    pmean = p[U_TOP:U_TOP + 1, :]

    corr = _dot(sel, p - pmean, dimension_numbers=(((0,), (0,)), ((), ())))

    @pl.when(h == 0)
    def _():
        out_ref[...] = corr[None]
        base_s[...] = pmean

    @pl.when(h > 0)
    def _():
        out_ref[...] += corr[None]
        base_s[...] += pmean

    @pl.when(h == N_HEADS - 1)
    def _():
        out_ref[...] += (base_s[...] + bo_ref[...])[None]


def kernel(query, key, value, W_Q, b_Q, W_K, b_K, W_V, b_V, W_O, b_O):
    k_N, Bq, d_model = query.shape

    cnt_t = jnp.asarray(_CNT_T_BF16)
    qb = query.transpose(1, 0, 2).astype(jnp.bfloat16)
    kb = key.transpose(1, 0, 2).astype(jnp.bfloat16)
    vb = value.transpose(1, 0, 2).astype(jnp.bfloat16)
    wqb = W_Q.reshape(N_HEADS, HEAD_DIM, d_model).astype(jnp.bfloat16)
    wkb = W_K.reshape(N_HEADS, HEAD_DIM, d_model).astype(jnp.bfloat16)
    wvb = W_V.reshape(N_HEADS, HEAD_DIM, d_model).astype(jnp.bfloat16)
    wob = W_O.T.reshape(N_HEADS, HEAD_DIM, d_model).astype(jnp.bfloat16)
    bq3 = b_Q.reshape(N_HEADS, 1, HEAD_DIM)
    bk3 = b_K.reshape(N_HEADS, 1, HEAD_DIM)
    bv3 = b_V.reshape(N_HEADS, 1, HEAD_DIM)
    bo2 = b_O.reshape(1, d_model)

    out = pl.pallas_call(
        _body,
        grid=(Bq, N_HEADS),
        in_specs=[
            pl.BlockSpec((1, K_N, D_MODEL), lambda b, h: (b, 0, 0)),
            pl.BlockSpec((1, K_N, D_MODEL), lambda b, h: (b, 0, 0)),
            pl.BlockSpec((1, K_N, D_MODEL), lambda b, h: (b, 0, 0)),
            pl.BlockSpec((1, HEAD_DIM, D_MODEL), lambda b, h: (h, 0, 0)),
            pl.BlockSpec((1, HEAD_DIM, D_MODEL), lambda b, h: (h, 0, 0)),
            pl.BlockSpec((1, HEAD_DIM, D_MODEL), lambda b, h: (h, 0, 0)),
            pl.BlockSpec((1, 1, HEAD_DIM), lambda b, h: (h, 0, 0)),
            pl.BlockSpec((1, 1, HEAD_DIM), lambda b, h: (h, 0, 0)),
            pl.BlockSpec((1, 1, HEAD_DIM), lambda b, h: (h, 0, 0)),
            pl.BlockSpec((1, HEAD_DIM, D_MODEL), lambda b, h: (h, 0, 0)),
            pl.BlockSpec((1, D_MODEL), lambda b, h: (0, 0)),
            pl.BlockSpec((K_N, K_N), lambda b, h: (0, 0)),
        ],
        out_specs=pl.BlockSpec((1, K_N, D_MODEL), lambda b, h: (b, 0, 0)),
        out_shape=jax.ShapeDtypeStruct((Bq, k_N, d_model), jnp.float32),
        scratch_shapes=[
            pltpu.VMEM((K_N, HEAD_DIM), jnp.float32),
            pltpu.VMEM((K_N, HEAD_DIM), jnp.bfloat16),
            pltpu.VMEM((K_N, HEAD_DIM), jnp.float32),
            pltpu.VMEM((1, K_N), jnp.float32),
            pltpu.VMEM((1, D_MODEL), jnp.float32),
        ],
    )(qb, kb, vb, wqb, wkb, wvb, bq3, bk3, bv3, wob, bo2, cnt_t)
    return out
